# baseline (speedup 1.0000x reference)
import functools

import jax
import jax.numpy as jnp
import numpy as np
from jax import lax
from jax.experimental import pallas as pl
from jax.experimental.pallas import tpu as pltpu
from jax.experimental.pallas import tpu_sc as plsc

_NC_SEGMENTS = 16000


def _pick_rows(n):
    for r in (800, 640, 512, 480, 400, 320, 256, 200, 160, 128, 80, 64, 40, 32, 16, 8):
        if n % r == 0:
            return r
    return n


def _fused_mm(xs, W, out_widths, add=None, affine=None, stats=False,
              splits=None, bmap=None):
    rows = xs[0].shape[0]
    K, N = W.shape
    kis = [x.shape[1] for x in xs]
    assert sum(kis) == K and sum(out_widths) == N
    if splits is None:
        splits = [[k] for k in kis]
    subs = []
    k_off = 0
    for j, sp in enumerate(splits):
        assert sum(sp) == kis[j]
        c = 0
        for w in sp:
            subs.append((j, c, w, k_off))
            c += w
            k_off += w
    if bmap is None:
        bmap = [[1] * len(out_widths) for _ in subs]
    R = _pick_rows(rows)
    G = rows // R
    n_x = len(xs)
    n_out = len(out_widths)
    w0 = out_widths[0]
    offs = np.cumsum([0] + list(out_widths))
    add_w = add.shape[1] if add is not None else 0

    def body(*refs):
        i = n_x
        xr = refs[:n_x]
        wr = refs[i]; i += 1
        if affine is not None:
            sr, hr = refs[i], refs[i + 1]; i += 2
        if add is not None:
            ar = refs[i]; i += 1
        outr = refs[i:i + n_out]; i += n_out
        if stats:
            sumr, sqr = refs[i], refs[i + 1]
        xbs = []
        k0 = 0
        for j in range(n_x):
            xb = xr[j][...]
            if affine is not None:
                xb = jnp.maximum(xb * sr[...][:, k0:k0 + kis[j]]
                                 + hr[...][:, k0:k0 + kis[j]], 0.0)
            xbs.append(xb)
            k0 += kis[j]
        if add is not None:
            ab = ar[...]
        for pidx in range(n_out):
            o0, o1 = int(offs[pidx]), int(offs[pidx + 1])
            v = None
            for si, (j, c0_, w_, ko) in enumerate(subs):
                kind = bmap[si][pidx]
                if kind == 0:
                    continue
                sub = xbs[j][:, c0_:c0_ + w_]
                if kind == 2:
                    d = sub
                else:
                    d = jnp.dot(sub, wr[ko:ko + w_, o0:o1],
                                preferred_element_type=jnp.float32)
                v = d if v is None else v + d
            if add is not None and o1 <= add_w:
                v = v + ab[:, o0:o1]
            outr[pidx][...] = v
            if stats and pidx == 0:
                ps = jnp.sum(v, axis=0, keepdims=True)
                pq = jnp.sum(v * v, axis=0, keepdims=True)

                @pl.when(pl.program_id(0) == 0)
                def _():
                    sumr[...] = ps
                    sqr[...] = pq

                @pl.when(pl.program_id(0) != 0)
                def _():
                    sumr[...] += ps
                    sqr[...] += pq

    in_specs = [pl.BlockSpec((R, k), lambda i: (i, 0)) for k in kis]
    in_specs.append(pl.BlockSpec((K, N), lambda i: (0, 0)))
    args = list(xs) + [W]
    if affine is not None:
        in_specs += [pl.BlockSpec((1, K), lambda i: (0, 0))] * 2
        args += [affine[0], affine[1]]
    if add is not None:
        in_specs.append(pl.BlockSpec((R, add_w), lambda i: (i, 0)))
        args.append(add)
    out_shape = [jax.ShapeDtypeStruct((rows, w), jnp.float32) for w in out_widths]
    out_specs = [pl.BlockSpec((R, w), lambda i: (i, 0)) for w in out_widths]
    if stats:
        out_shape += [jax.ShapeDtypeStruct((1, w0), jnp.float32)] * 2
        out_specs += [pl.BlockSpec((1, w0), lambda i: (0, 0))] * 2
    return pl.pallas_call(
        body, grid=(G,), in_specs=in_specs, out_specs=out_specs,
        out_shape=out_shape,
        compiler_params=pltpu.CompilerParams(
            dimension_semantics=("arbitrary",)),
    )(*args)


def _ew_affine_relu(x, scale, shift):
    rows, w = x.shape
    R = _pick_rows(rows)

    def body(xr, sr, hr, outr):
        outr[...] = jnp.maximum(xr[...] * sr[...] + hr[...], 0.0)

    return pl.pallas_call(
        body, grid=(rows // R,),
        in_specs=[pl.BlockSpec((R, w), lambda i: (i, 0)),
                  pl.BlockSpec((1, w), lambda i: (0, 0)),
                  pl.BlockSpec((1, w), lambda i: (0, 0))],
        out_specs=pl.BlockSpec((R, w), lambda i: (i, 0)),
        out_shape=jax.ShapeDtypeStruct((rows, w), jnp.float32),
    )(x, scale, shift)


def _st_mm(seg, cscnt, WST, h):
    rows = seg.shape[0]
    R = _pick_rows(rows)

    def body(segr, cr, wr, outr):
        sB = segr[:, :h]
        sP = segr[:, h:2 * h]
        sQ = segr[:, 2 * h:3 * h]
        cs = cr[:, :h]
        n = cr[:, h:h + 1]
        x2 = sP + n * sQ
        acc = (jnp.dot(sQ, wr[0:h, :], preferred_element_type=jnp.float32)
               + jnp.dot(x2, wr[h:2 * h, :], preferred_element_type=jnp.float32)
               + jnp.dot(cs, wr[2 * h:3 * h, :], preferred_element_type=jnp.float32)
               + jnp.dot(sB, wr[3 * h:4 * h, :], preferred_element_type=jnp.float32))
        outr[...] = acc

    return pl.pallas_call(
        body, grid=(rows // R,),
        in_specs=[pl.BlockSpec((R, seg.shape[1]), lambda i: (i, 0)),
                  pl.BlockSpec((R, cscnt.shape[1]), lambda i: (i, 0)),
                  pl.BlockSpec(WST.shape, lambda i: (0, 0))],
        out_specs=pl.BlockSpec((R, WST.shape[1]), lambda i: (i, 0)),
        out_shape=jax.ShapeDtypeStruct((rows, WST.shape[1]), jnp.float32),
    )(seg, cscnt, WST)


def _sc_gather(table, idx):
    mi = idx.shape[0]
    wrow = table.shape[1]
    nw = 32
    per_w = mi // nw
    chunk = 120
    assert per_w % chunk == 0 and chunk % 8 == 0 and per_w % 8 == 0
    nch = per_w // chunk
    mesh = plsc.VectorSubcoreMesh(core_axis_name="c", subcore_axis_name="s", num_cores=2, num_subcores=16)

    @functools.partial(
        pl.kernel, mesh=mesh,
        out_type=jax.ShapeDtypeStruct((mi, wrow), jnp.float32),
        scratch_types=[
            pltpu.VMEM((chunk,), jnp.int32),
            pltpu.VMEM((chunk, wrow), jnp.float32),
            pltpu.SemaphoreType.DMA,
        ],
    )
    def k(table_hbm, idx_hbm, out_hbm, idx_v, rows_v, sem):
        wid = lax.axis_index("s") * 2 + lax.axis_index("c")

        def step(j, carry):
            base = pl.multiple_of(wid * per_w + j * chunk, 8)
            pltpu.sync_copy(idx_hbm.at[pl.ds(base, chunk)], idx_v)
            pltpu.async_copy(table_hbm.at[idx_v], rows_v, sem).wait()
            pltpu.sync_copy(rows_v, out_hbm.at[pl.ds(base, chunk)])
            return carry

        lax.fori_loop(0, nch, step, 0)

    return k(table, idx)


def _sc_scatter_add(values, idx, t_pad, wb):
    mi = idx.shape[0]
    widths = [v.shape[1] for v in values]
    wtot = sum(widths)
    nblk = wtot // wb
    assert wtot % wb == 0 and nblk % 2 == 0 and t_pad % 128 == 0
    per_tile = mi // 16
    chunk = 1000
    assert per_tile % chunk == 0 and per_tile % 8 == 0
    nch = per_tile // chunk
    tr = t_pad // 16
    col_bounds = np.cumsum([0] + widths)
    zeros = jnp.zeros((tr, wb), jnp.float32)
    mesh = plsc.VectorSubcoreMesh(core_axis_name="c", subcore_axis_name="s", num_cores=2, num_subcores=16)

    @functools.partial(
        pl.kernel, mesh=mesh,
        out_type=jax.ShapeDtypeStruct((t_pad, wtot), jnp.float32),
        scratch_types=[
            pltpu.VMEM((chunk, wb), jnp.float32),
            pltpu.VMEM((chunk,), jnp.int32),
            pltpu.VMEM_SHARED((t_pad, wb), jnp.float32),
        ],
        compiler_params=pltpu.CompilerParams(use_tc_tiling_on_sc=False),
    )
    def k(*refs):
        nv = len(values)
        vals_hbm = refs[:nv]
        idx_hbm = refs[nv]
        z_hbm = refs[nv + 1]
        out_hbm = refs[nv + 2]
        vbuf, ibuf, shared = refs[nv + 3:]
        core = lax.axis_index("c")
        sid = lax.axis_index("s")
        for blk in range(nblk):
            c0 = blk * wb
            ai = int(np.searchsorted(col_bounds, c0, side="right") - 1)
            src = vals_hbm[ai]
            coff = c0 - int(col_bounds[ai])
            active = (blk % 2) == core

            @pl.when(active)
            def _init():
                pltpu.sync_copy(z_hbm, shared.at[pl.ds(sid * tr, tr)])

            plsc.subcore_barrier()

            @pl.when(active)
            def _scat():
                def step(ch, carry):
                    base = pl.multiple_of(sid * per_tile + ch * chunk, 8)
                    pltpu.sync_copy(idx_hbm.at[pl.ds(base, chunk)], ibuf)
                    pltpu.sync_copy(
                        src.at[pl.ds(base, chunk), pl.ds(coff, wb)], vbuf)
                    pltpu.sync_copy(vbuf, shared.at[ibuf], add=True)
                    return carry

                lax.fori_loop(0, nch, step, 0)

            plsc.subcore_barrier()

            @pl.when(active)
            def _wout():
                pltpu.sync_copy(
                    shared.at[pl.ds(sid * tr, tr)],
                    out_hbm.at[pl.ds(sid * tr, tr), pl.ds(c0, wb)])

            plsc.subcore_barrier()

    return k(*values, idx, zeros)


def kernel(edge_rep, cycle_rep, params, edge_idx, cycle_ids):
    p = params
    ne, h = edge_rep.shape
    m = cycle_ids.shape[0]
    nc = _NC_SEGMENTS
    f32 = jnp.float32
    ei = edge_idx.astype(jnp.int32)
    ci = cycle_ids.astype(jnp.int32)
    Z = jnp.zeros((h, h), f32)

    W1i, W1v = p['mlp1_int_W1'], p['mlp1_inv_W1']
    W1x_i, W1y_i, W1z_i = W1i[:h], W1i[h:2 * h], W1i[2 * h:]
    W1x_v, W1y_v, W1z_v = W1v[:h], W1v[h:2 * h], W1v[2 * h:]
    Li, Lv = p['lift_lin_int'], p['lift_lin_inv']
    L1, L2 = p['lvl_aggr_lin'][:h], p['lvl_aggr_lin'][h:]
    Wa, Wb, Wc = p['lift_W1'][:h], p['lift_W1'][h:2 * h], p['lift_W1'][2 * h:]
    W2i, W2v = p['mlp1_int_W2'], p['mlp1_inv_W2']

    def bn_affine(s, q, rows, g, b):
        mu = s / rows
        var = q / rows - mu * mu
        rs = lax.rsqrt(var + 1e-5)
        scale = rs * g[None, :]
        shift = b[None, :] - mu * scale
        return scale, shift

    x = _sc_gather(edge_rep, ei)
    ones128 = jnp.ones((m, 128), f32)
    cscnt = _sc_scatter_add([cycle_rep, ones128], ci, nc, 64)
    Wz = jnp.concatenate(
        [jnp.concatenate([W1z_i, W1z_v], 1), jnp.zeros((128, 2 * h), f32)], 0)
    (Ucat,) = _fused_mm([cscnt], Wz, [2 * h])
    G1 = _sc_gather(Ucat, ci)

    Wbig = jnp.block([[W1x_i, W1x_v, Li, Lv],
                      [W1y_i, W1y_v, Z, Z]])
    Z12, A, B, s1, q1 = _fused_mm(
        [x, cycle_rep], Wbig, [2 * h, h, h], add=G1, stats=True,
        bmap=[[1, 1, 1], [1, 0, 0]])
    g12 = jnp.concatenate([p['mlp1_int_bn_g'], p['mlp1_inv_bn_g']])
    b12 = jnp.concatenate([p['mlp1_int_bn_b'], p['mlp1_inv_bn_b']])
    sc12, sh12 = bn_affine(s1, q1, m, g12, b12)

    W2big = jnp.block([[W2i, Z, W2i @ L1],
                       [Z, W2v, Z]])
    P, Q, R = _fused_mm([Z12], W2big, [h, h, h], affine=(sc12, sh12),
                        splits=[[h, h]], bmap=[[1, 0, 1], [0, 1, 0]])

    seg = _sc_scatter_add([B, P, Q], ci, nc, 64)
    WST = jnp.block([[Z, L1],
                     [Z, L2],
                     [Wb, Z],
                     [Wc, Z]])
    TS = _st_mm(seg, cscnt, WST, h)
    G2 = _sc_gather(TS, ci)

    WzcR = jnp.block([[Wa, Z],
                      [Wc, Z],
                      [Z, jnp.eye(h, dtype=f32)]])
    zc, contrib, s_c, q_c = _fused_mm(
        [cycle_rep, A, R], WzcR, [h, h], add=G2, stats=True,
        bmap=[[1, 0], [1, 0], [0, 2]])

    ne_pad = ((ne + 127) // 128) * 128
    lvl = _sc_scatter_add([contrib], ei, ne_pad, 16)[:ne]

    W13 = jnp.concatenate([(1.0 + p['eps']) * p['mlp2_W1'], p['mlp2_W1']], 0)
    z1, s_e1, q_e1 = _fused_mm([edge_rep, lvl], W13, [h], stats=True)
    sc_e1, sh_e1 = bn_affine(s_e1, q_e1, ne, p['mlp2_bn1_g'], p['mlp2_bn1_b'])
    z2, s_e2, q_e2 = _fused_mm([z1], p['mlp2_W2'], [h],
                               affine=(sc_e1, sh_e1), stats=True)
    sc_e2, sh_e2 = bn_affine(s_e2, q_e2, ne, p['mlp2_bn2_g'], p['mlp2_bn2_b'])
    edge_out = _ew_affine_relu(z2, sc_e2, sh_e2)

    sc_c1, sh_c1 = bn_affine(s_c, q_c, m, p['lift_bn1_g'], p['lift_bn1_b'])
    zc2, s_c2, q_c2 = _fused_mm([zc], p['lift_W2'], [h],
                                affine=(sc_c1, sh_c1), stats=True)
    sc_c2, sh_c2 = bn_affine(s_c2, q_c2, m, p['lift_bn2_g'], p['lift_bn2_b'])
    cycle_out = _ew_affine_relu(zc2, sc_c2, sh_c2)
    return edge_out, cycle_out
